# baseline (speedup 1.0000x reference)
import functools
import math

import jax
import jax.numpy as jnp
from jax import lax
from jax.experimental import pallas as pl
from jax.experimental.pallas import tpu as pltpu
from jax.experimental.pallas import tpu_sc as plsc

N_FEAT = 256
N_ROWS = 32768
NW = 32
SC_ROWS = 4096
TC_ROWS = N_ROWS - SC_ROWS
ROWS_PER_W = SC_ROWS // NW
CHUNK = 64
NCHUNK = ROWS_PER_W // CHUNK
CHUNK_ELEMS = CHUNK * N_FEAT
N_GROUPS = N_FEAT // 16
ROW_BLOCK = 2048
TC_BLOCK0 = SC_ROWS // ROW_BLOCK


def _coeffs(angles, indices_in, idx_out):
    n_in = angles.shape[0] // 2
    j = jnp.arange(N_FEAT, dtype=jnp.int32)

    def stage(idx, th):
        pa, pb = idx[0::2], idx[1::2]
        c_h, s_h = jnp.cos(th), jnp.sin(th)
        a = (pa[:, None] == j[None, :]).astype(jnp.float32)
        b = (pb[:, None] == j[None, :]).astype(jnp.float32)
        hp = jax.lax.Precision.HIGHEST
        c = jnp.dot(c_h, a, precision=hp) + jnp.dot(c_h, b, precision=hp)
        s = jnp.dot(s_h, a, precision=hp) - jnp.dot(s_h, b, precision=hp)
        return c, s

    ca, sa = stage(indices_in, jnp.sum(angles[:n_in], axis=0))
    cb, sb = stage(idx_out, jnp.sum(angles[n_in:], axis=0))
    return ca, sa, cb, sb


def _combined_matrix(ca, sa, cb, sb):
    i = jnp.arange(N_FEAT, dtype=jnp.int32)
    p1 = (i[:, None] == (i ^ 1)[None, :]).astype(jnp.float32)
    p2 = (i[:, None] == (i ^ 128)[None, :]).astype(jnp.float32)
    m_in = jnp.diag(ca) + sa[None, :] * p1
    m_out = jnp.diag(cb) + sb[None, :] * p2
    return jnp.dot(m_in, m_out, precision=jax.lax.Precision.HIGHEST)


def _sc_body(data_hbm, ca_hbm, sa_hbm, cb_hbm, sb_hbm, out_hbm,
             x0, x1, o0, o1, ca_v, sa_v, cb_v, sb_v,
             si0, si1, so0, so1):
    wid = lax.axis_index("s") * 2 + lax.axis_index("c")
    base = wid * (ROWS_PER_W * N_FEAT)

    pltpu.sync_copy(ca_hbm, ca_v)
    pltpu.sync_copy(sa_hbm, sa_v)
    pltpu.sync_copy(cb_hbm, cb_v)
    pltpu.sync_copy(sb_hbm, sb_v)

    xbufs = (x0, x1)
    obufs = (o0, o1)
    isems = (si0, si1)
    osems = (so0, so1)
    perm = lax.iota(jnp.int32, 16) ^ 1

    def copy_in(c, buf, sem):
        off = base + c * CHUNK_ELEMS
        pltpu.make_async_copy(
            data_hbm.at[pl.ds(off, CHUNK_ELEMS)], buf, sem).start()

    def copy_out(c, buf, sem):
        off = base + c * CHUNK_ELEMS
        pltpu.make_async_copy(
            buf, out_hbm.at[pl.ds(off, CHUNK_ELEMS)], sem).start()

    copy_in(0, x0, si0)

    def do_pair(cc, _):
        for b in range(2):
            c = cc * 2 + b
            xb, ob = xbufs[b], obufs[b]

            @pl.when(c + 1 < NCHUNK)
            def _():
                copy_in(c + 1, xbufs[1 - b], isems[1 - b])

            pltpu.make_async_copy(
                data_hbm.at[pl.ds(0, CHUNK_ELEMS)], xb, isems[b]).wait()

            @pl.when(c >= 2)
            def _():
                pltpu.make_async_copy(
                    ob, out_hbm.at[pl.ds(0, CHUNK_ELEMS)], osems[b]).wait()

            for g in range(N_GROUPS // 2):
                slg = pl.ds(g * 16, 16)
                slh = pl.ds((g + 8) * 16, 16)
                cag, sag = ca_v[slg], sa_v[slg]
                cah, sah = ca_v[slh], sa_v[slh]
                cbg, sbg = cb_v[slg], sb_v[slg]
                cbh, sbh = cb_v[slh], sb_v[slh]

                @plsc.parallel_loop(0, CHUNK, step=1, unroll=4)
                def do_row(r, g=g, cag=cag, sag=sag, cah=cah, sah=sah,
                           cbg=cbg, sbg=sbg, cbh=cbh, sbh=sbh):
                    rbase = r * N_FEAT
                    xg = xb[pl.ds(rbase + g * 16, 16)]
                    xh = xb[pl.ds(rbase + (g + 8) * 16, 16)]
                    ya = cag * xg + sag * xg[perm]
                    yb = cah * xh + sah * xh[perm]
                    ob[pl.ds(rbase + g * 16, 16)] = cbg * ya + sbg * yb
                    ob[pl.ds(rbase + (g + 8) * 16, 16)] = cbh * yb + sbh * ya

            copy_out(c, ob, osems[b])
        return 0

    lax.fori_loop(0, NCHUNK // 2, do_pair, 0)
    pltpu.make_async_copy(
        o0, out_hbm.at[pl.ds(0, CHUNK_ELEMS)], so0).wait()
    pltpu.make_async_copy(
        o1, out_hbm.at[pl.ds(0, CHUNK_ELEMS)], so1).wait()


def _sc_call(data_flat, ca, sa, cb, sb):
    mesh = plsc.VectorSubcoreMesh(core_axis_name="c", subcore_axis_name="s")
    k = functools.partial(
        pl.kernel,
        mesh=mesh,
        compiler_params=pltpu.CompilerParams(
            use_tc_tiling_on_sc=False, needs_layout_passes=False
        ),
        out_type=jax.ShapeDtypeStruct((N_ROWS * N_FEAT,), jnp.float32),
        scratch_types=[
            pltpu.VMEM((CHUNK_ELEMS,), jnp.float32),
            pltpu.VMEM((CHUNK_ELEMS,), jnp.float32),
            pltpu.VMEM((CHUNK_ELEMS,), jnp.float32),
            pltpu.VMEM((CHUNK_ELEMS,), jnp.float32),
            pltpu.VMEM((N_FEAT,), jnp.float32),
            pltpu.VMEM((N_FEAT,), jnp.float32),
            pltpu.VMEM((N_FEAT,), jnp.float32),
            pltpu.VMEM((N_FEAT,), jnp.float32),
            pltpu.SemaphoreType.DMA,
            pltpu.SemaphoreType.DMA,
            pltpu.SemaphoreType.DMA,
            pltpu.SemaphoreType.DMA,
        ],
    )(_sc_body)
    return k(data_flat, ca, sa, cb, sb)


def _tc_matmul_kernel(sc_ref, x_ref, w_ref, o_ref):
    o_ref[...] = jnp.dot(
        x_ref[...],
        w_ref[...],
        preferred_element_type=jnp.float32,
        precision=jax.lax.Precision.DEFAULT,
    )


def _tc_call(sc_full, data, w):
    grid = (TC_ROWS // ROW_BLOCK,)
    return pl.pallas_call(
        _tc_matmul_kernel,
        grid=grid,
        in_specs=[
            pl.BlockSpec((8, N_FEAT), lambda i: (0, 0)),
            pl.BlockSpec((ROW_BLOCK, N_FEAT), lambda i: (i + TC_BLOCK0, 0)),
            pl.BlockSpec((N_FEAT, N_FEAT), lambda i: (0, 0)),
        ],
        out_specs=pl.BlockSpec((ROW_BLOCK, N_FEAT), lambda i: (i + TC_BLOCK0, 0)),
        out_shape=jax.ShapeDtypeStruct((N_ROWS, N_FEAT), jnp.float32),
        input_output_aliases={0: 0},
    )(sc_full, data, w)


def kernel(data, angles, indices_in, idx_out):
    ca, sa, cb, sb = _coeffs(angles, indices_in, idx_out)
    w = _combined_matrix(ca, sa, cb, sb)
    sc_out = _sc_call(data.reshape(-1), ca, sa, cb, sb)
    return _tc_call(sc_out.reshape(N_ROWS, N_FEAT), data, w)


# speedup vs baseline: 1.5282x; 1.5282x over previous
import functools
import math

import jax
import jax.numpy as jnp
from jax import lax
from jax.experimental import pallas as pl
from jax.experimental.pallas import tpu as pltpu
from jax.experimental.pallas import tpu_sc as plsc

N_FEAT = 256
N_ROWS = 32768
NW = 32
SC_ROWS = 4096
TC_ROWS = N_ROWS - SC_ROWS
ROWS_PER_W = SC_ROWS // NW
CHUNK = 64
NCHUNK = ROWS_PER_W // CHUNK
CHUNK_ELEMS = CHUNK * N_FEAT
N_GROUPS = N_FEAT // 16
ROW_BLOCK = 2048
TC_BLOCK0 = SC_ROWS // ROW_BLOCK


def _coeffs(angles, indices_in, idx_out):
    n_in = angles.shape[0] // 2
    j = jnp.arange(N_FEAT, dtype=jnp.int32)

    def stage(idx, th):
        pa, pb = idx[0::2], idx[1::2]
        c_h, s_h = jnp.cos(th), jnp.sin(th)
        a = (pa[:, None] == j[None, :]).astype(jnp.float32)
        b = (pb[:, None] == j[None, :]).astype(jnp.float32)
        hp = jax.lax.Precision.HIGHEST
        c = jnp.dot(c_h, a, precision=hp) + jnp.dot(c_h, b, precision=hp)
        s = jnp.dot(s_h, a, precision=hp) - jnp.dot(s_h, b, precision=hp)
        return c, s

    ca, sa = stage(indices_in, jnp.sum(angles[:n_in], axis=0))
    cb, sb = stage(idx_out, jnp.sum(angles[n_in:], axis=0))
    return ca, sa, cb, sb


def _combined_matrix(ca, sa, cb, sb):
    i = jnp.arange(N_FEAT, dtype=jnp.int32)
    p1 = (i[:, None] == (i ^ 1)[None, :]).astype(jnp.float32)
    p2 = (i[:, None] == (i ^ 128)[None, :]).astype(jnp.float32)
    m_in = jnp.diag(ca) + sa[None, :] * p1
    m_out = jnp.diag(cb) + sb[None, :] * p2
    return jnp.dot(m_in, m_out, precision=jax.lax.Precision.HIGHEST)


def _sc_body(data_hbm, ca_hbm, sa_hbm, cb_hbm, sb_hbm, out_hbm,
             x0, x1, o0, o1, ca_v, sa_v, cb_v, sb_v,
             si0, si1, so0, so1):
    wid = lax.axis_index("s") * 2 + lax.axis_index("c")
    base = wid * (ROWS_PER_W * N_FEAT)

    pltpu.sync_copy(ca_hbm, ca_v)
    pltpu.sync_copy(sa_hbm, sa_v)
    pltpu.sync_copy(cb_hbm, cb_v)
    pltpu.sync_copy(sb_hbm, sb_v)

    xbufs = (x0, x1)
    obufs = (o0, o1)
    isems = (si0, si1)
    osems = (so0, so1)
    perm = lax.iota(jnp.int32, 16) ^ 1

    def copy_in(c, buf, sem):
        off = base + c * CHUNK_ELEMS
        pltpu.make_async_copy(
            data_hbm.at[pl.ds(off, CHUNK_ELEMS)], buf, sem).start()

    def copy_out(c, buf, sem):
        off = base + c * CHUNK_ELEMS
        pltpu.make_async_copy(
            buf, out_hbm.at[pl.ds(off, CHUNK_ELEMS)], sem).start()

    copy_in(0, x0, si0)

    def do_pair(cc, _):
        for b in range(2):
            c = cc * 2 + b
            xb, ob = xbufs[b], obufs[b]

            @pl.when(c + 1 < NCHUNK)
            def _():
                copy_in(c + 1, xbufs[1 - b], isems[1 - b])

            pltpu.make_async_copy(
                data_hbm.at[pl.ds(0, CHUNK_ELEMS)], xb, isems[b]).wait()

            @pl.when(c >= 2)
            def _():
                pltpu.make_async_copy(
                    ob, out_hbm.at[pl.ds(0, CHUNK_ELEMS)], osems[b]).wait()

            for g in range(N_GROUPS // 2):
                slg = pl.ds(g * 16, 16)
                slh = pl.ds((g + 8) * 16, 16)
                cag, sag = ca_v[slg], sa_v[slg]
                cah, sah = ca_v[slh], sa_v[slh]
                cbg, sbg = cb_v[slg], sb_v[slg]
                cbh, sbh = cb_v[slh], sb_v[slh]

                @plsc.parallel_loop(0, CHUNK, step=1, unroll=4)
                def do_row(r, g=g, cag=cag, sag=sag, cah=cah, sah=sah,
                           cbg=cbg, sbg=sbg, cbh=cbh, sbh=sbh):
                    rbase = r * N_FEAT
                    xg = xb[pl.ds(rbase + g * 16, 16)]
                    xh = xb[pl.ds(rbase + (g + 8) * 16, 16)]
                    ya = cag * xg + sag * xg[perm]
                    yb = cah * xh + sah * xh[perm]
                    ob[pl.ds(rbase + g * 16, 16)] = cbg * ya + sbg * yb
                    ob[pl.ds(rbase + (g + 8) * 16, 16)] = cbh * yb + sbh * ya

            copy_out(c, ob, osems[b])
        return 0

    lax.fori_loop(0, NCHUNK // 2, do_pair, 0)
    pltpu.make_async_copy(
        o0, out_hbm.at[pl.ds(0, CHUNK_ELEMS)], so0).wait()
    pltpu.make_async_copy(
        o1, out_hbm.at[pl.ds(0, CHUNK_ELEMS)], so1).wait()


def _sc_call(data_flat, ca, sa, cb, sb):
    mesh = plsc.VectorSubcoreMesh(core_axis_name="c", subcore_axis_name="s")
    k = functools.partial(
        pl.kernel,
        mesh=mesh,
        compiler_params=pltpu.CompilerParams(
            use_tc_tiling_on_sc=False, needs_layout_passes=False
        ),
        out_type=jax.ShapeDtypeStruct((SC_ROWS * N_FEAT,), jnp.float32),
        scratch_types=[
            pltpu.VMEM((CHUNK_ELEMS,), jnp.float32),
            pltpu.VMEM((CHUNK_ELEMS,), jnp.float32),
            pltpu.VMEM((CHUNK_ELEMS,), jnp.float32),
            pltpu.VMEM((CHUNK_ELEMS,), jnp.float32),
            pltpu.VMEM((N_FEAT,), jnp.float32),
            pltpu.VMEM((N_FEAT,), jnp.float32),
            pltpu.VMEM((N_FEAT,), jnp.float32),
            pltpu.VMEM((N_FEAT,), jnp.float32),
            pltpu.SemaphoreType.DMA,
            pltpu.SemaphoreType.DMA,
            pltpu.SemaphoreType.DMA,
            pltpu.SemaphoreType.DMA,
        ],
    )(_sc_body)
    return k(data_flat, ca, sa, cb, sb)


def _tc_matmul_kernel(x_ref, w_ref, o_ref):
    o_ref[...] = jnp.dot(
        x_ref[...],
        w_ref[...],
        preferred_element_type=jnp.float32,
        precision=jax.lax.Precision.DEFAULT,
    )


def _tc_call(data, w):
    grid = (TC_ROWS // ROW_BLOCK,)
    return pl.pallas_call(
        _tc_matmul_kernel,
        grid=grid,
        in_specs=[
            pl.BlockSpec((ROW_BLOCK, N_FEAT), lambda i: (i + TC_BLOCK0, 0)),
            pl.BlockSpec((N_FEAT, N_FEAT), lambda i: (0, 0)),
        ],
        out_specs=pl.BlockSpec((ROW_BLOCK, N_FEAT), lambda i: (i + TC_BLOCK0, 0)),
        out_shape=jax.ShapeDtypeStruct((N_ROWS, N_FEAT), jnp.float32),
    )(data, w)


def kernel(data, angles, indices_in, idx_out):
    ca, sa, cb, sb = _coeffs(angles, indices_in, idx_out)
    w = _combined_matrix(ca, sa, cb, sb)
    sc_out = _sc_call(data.reshape(-1), ca, sa, cb, sb)
    tc_out = _tc_call(data, w)
    return lax.dynamic_update_slice(
        tc_out, sc_out.reshape(SC_ROWS, N_FEAT), (0, 0))
